# baseline (speedup 1.0000x reference)
import functools

import jax
import jax.numpy as jnp
from jax import lax
from jax.experimental import pallas as pl
from jax.experimental.pallas import tpu as pltpu
from jax.experimental.pallas import tpu_sc as plsc

FIELD = 26
BATCH = 16384
LANES = 16
NC = 2
NS = 16
NW = NC * NS
ROWS_W = BATCH // NW
GROUPS = ROWS_W // LANES
FLAT = FIELD * ROWS_W


def _sc_body(ids_hbm, vals_hbm, w_hbm, b_hbm, out_hbm,
             idx_v, vals_v, g_v, acc_v, out_v, b_v, zidx_v, sems, bsem):
    c = lax.axis_index("c")
    s = lax.axis_index("s")
    wid = s * NC + c

    pltpu.sync_copy(ids_hbm.at[wid], idx_v)
    pltpu.sync_copy(vals_hbm.at[wid], vals_v)
    zidx_v[...] = jnp.zeros((LANES,), jnp.int32)
    pltpu.async_copy(b_hbm.at[zidx_v], b_v, bsem).wait()
    for j in range(FIELD):
        pltpu.async_copy(
            w_hbm.at[idx_v.at[pl.ds(j * ROWS_W, ROWS_W)]],
            g_v.at[pl.ds(j * ROWS_W, ROWS_W)],
            sems.at[j])

    zero = jnp.zeros((LANES,), jnp.float32)
    for t in range(GROUPS):
        acc_v[pl.ds(t * LANES, LANES)] = zero

    def field(j, carry):
        off = j * ROWS_W
        pltpu.make_async_copy(
            w_hbm.at[idx_v.at[pl.ds(off, ROWS_W)]],
            g_v.at[pl.ds(off, ROWS_W)],
            sems.at[j]).wait()
        for t in range(GROUPS):
            w = g_v[pl.ds(off + t * LANES, LANES)]
            v = vals_v[pl.ds(off + t * LANES, LANES)]
            plsc.addupdate(acc_v.at[pl.ds(t * LANES, LANES)], w * v)
        return carry

    lax.fori_loop(0, FIELD, field, 0)

    bias = b_v[...]
    for t in range(GROUPS):
        z = acc_v[pl.ds(t * LANES, LANES)] + bias
        out_v[pl.ds(t * LANES, LANES)] = 1.0 / (1.0 + jnp.exp(-z))
    pltpu.sync_copy(out_v, out_hbm.at[pl.ds(wid * ROWS_W, ROWS_W)])


_sc_kernel = functools.partial(
    pl.kernel,
    out_type=jax.ShapeDtypeStruct((BATCH,), jnp.float32),
    mesh=plsc.VectorSubcoreMesh(core_axis_name="c", subcore_axis_name="s"),
    compiler_params=pltpu.CompilerParams(needs_layout_passes=False),
    scratch_types=[
        pltpu.VMEM((FLAT,), jnp.int32),
        pltpu.VMEM((FLAT,), jnp.float32),
        pltpu.VMEM((FLAT,), jnp.float32),
        pltpu.VMEM((ROWS_W,), jnp.float32),
        pltpu.VMEM((ROWS_W,), jnp.float32),
        pltpu.VMEM((LANES,), jnp.float32),
        pltpu.VMEM((LANES,), jnp.int32),
        pltpu.SemaphoreType.DMA((FIELD,)),
        pltpu.SemaphoreType.DMA,
    ],
)(_sc_body)


def kernel(feat_ids, feat_vals, LR_W, LR_B):
    ids_t = feat_ids.reshape(NW, ROWS_W, FIELD).transpose(0, 2, 1).reshape(NW, FLAT)
    vals_t = feat_vals.reshape(NW, ROWS_W, FIELD).transpose(0, 2, 1).reshape(NW, FLAT)
    return _sc_kernel(ids_t, vals_t, LR_W, LR_B)
